# baseline (speedup 1.0000x reference)
import functools

import jax
import jax.numpy as jnp
from jax import lax
from jax.experimental import pallas as pl
from jax.experimental.pallas import tpu as pltpu
from jax.experimental.pallas import tpu_sc as plsc

N_ATOM = 10000
N_BOND = 10000
N_GLOBAL = 64
E_BOND = 320000
E_GLOBAL = 10000
D = 128
DW = D + 16
ACC_ROWS = 10112
NC, NS = 2, 16
NW = NC * NS
CHUNK = 128
NCH_C0 = 124
NCH_C1 = 34
ROWS_PER_SUB = ACC_ROWS // NS


def _ceil_div(a, b):
    return (a + b - 1) // b


def _even_chunks(n_edges):
    n = _ceil_div(n_edges, NW * CHUNK)
    return n + (n % 2)


@functools.lru_cache(maxsize=None)
def _make_segsum(nch1a, nch1b, nch2):
    mesh = plsc.VectorSubcoreMesh(core_axis_name="c", subcore_axis_name="s")

    @functools.partial(
        pl.kernel,
        out_type=(jax.ShapeDtypeStruct((NC, ACC_ROWS, DW), jnp.float32),
                  jax.ShapeDtypeStruct((1, ACC_ROWS, DW), jnp.float32)),
        mesh=mesh,
        scratch_types=[
            pltpu.VMEM((2, CHUNK), jnp.int32),
            pltpu.VMEM((2, CHUNK), jnp.int32),
            pltpu.VMEM((CHUNK, DW), jnp.float32),
            pltpu.VMEM((CHUNK, DW), jnp.float32),
            pltpu.VMEM_SHARED((ACC_ROWS, DW), jnp.float32),
            pltpu.SemaphoreType.DMA,
            pltpu.SemaphoreType.DMA,
            pltpu.SemaphoreType.DMA,
        ],
        compiler_params=pltpu.CompilerParams(use_tc_tiling_on_sc=False),
    )
    def segsum(tab1_hbm, tab2_hbm, slab1_hbm, slab2_hbm, zeros_hbm,
               out1_hbm, out2_hbm,
               idx_a, idx_b, rows_a, rows_b, acc, sem_a, sem_b, sem_ib):
        cid = lax.axis_index("c")
        sid = lax.axis_index("s")
        r0 = sid * ROWS_PER_SUB

        def phase(table_hbm, slab_hbm, out_hbm, n_chunks, out_idx, base):
            pltpu.sync_copy(zeros_hbm.at[pl.ds(r0, ROWS_PER_SUB)],
                            acc.at[pl.ds(r0, ROWS_PER_SUB)])
            plsc.subcore_barrier()
            pltpu.sync_copy(slab_hbm.at[base], idx_a)
            pltpu.async_copy(table_hbm.at[idx_a.at[0]], rows_a, sem_a)
            pltpu.async_copy(slab_hbm.at[base + 1], idx_b, sem_ib)

            @pl.loop(0, n_chunks, step=2)
            def _(j):
                pltpu.make_async_copy(slab_hbm.at[base + j + 1], idx_b,
                                      sem_ib).wait()
                pltpu.async_copy(table_hbm.at[idx_b.at[0]], rows_b, sem_b)
                pltpu.make_async_copy(table_hbm.at[idx_a.at[0]], rows_a,
                                      sem_a).wait()
                pltpu.sync_copy(rows_a, acc.at[idx_a.at[1]], add=True)

                @pl.when(j + 2 < n_chunks)
                def _():
                    pltpu.sync_copy(slab_hbm.at[base + j + 2], idx_a)
                    pltpu.async_copy(table_hbm.at[idx_a.at[0]], rows_a, sem_a)

                pltpu.make_async_copy(table_hbm.at[idx_b.at[0]], rows_b,
                                      sem_b).wait()
                pltpu.sync_copy(rows_b, acc.at[idx_b.at[1]], add=True)

                @pl.when(j + 3 < n_chunks)
                def _():
                    pltpu.async_copy(slab_hbm.at[base + j + 3], idx_b, sem_ib)

            plsc.subcore_barrier()
            pltpu.sync_copy(acc.at[pl.ds(r0, ROWS_PER_SUB)],
                            out_hbm.at[out_idx].at[pl.ds(r0, ROWS_PER_SUB)])

        @pl.when(cid == 0)
        def _():
            phase(tab1_hbm, slab1_hbm, out1_hbm, nch1a, 0, sid * nch1a)
            phase(tab2_hbm, slab2_hbm, out2_hbm, nch2, 0, sid * nch2)

        @pl.when(cid == 1)
        def _():
            phase(tab1_hbm, slab1_hbm, out1_hbm, nch1b, 1,
                  NS * nch1a + sid * nch1b)

    return segsum


def _edge_chunks(src, dst, n_chunks_total):
    pad = n_chunks_total * CHUNK - src.shape[0]
    dummy = N_ATOM + jnp.arange(pad, dtype=jnp.int32) % (ACC_ROWS - N_ATOM)
    src_p = jnp.concatenate([src, jnp.zeros((pad,), jnp.int32)])
    dst_p = jnp.concatenate([dst, dummy])
    return jnp.stack([src_p.reshape(-1, CHUNK),
                      dst_p.reshape(-1, CHUNK)], axis=1)


BLK = 2000


def _mlp_body(master_ref, p1_hbm, p2_hbm, w1_ref, b1_ref, w2_ref, b2_ref,
              w3_ref, b3_ref, out_ref, p1_v, p2_v, sem1, sem2):
    i = pl.program_id(0)
    c1 = pltpu.make_async_copy(p1_hbm.at[:, pl.ds(i * BLK, BLK), :],
                               p1_v, sem1)
    c2 = pltpu.make_async_copy(p2_hbm.at[:, pl.ds(i * BLK, BLK), :],
                               p2_v, sem2)
    c1.start()
    c2.start()
    c1.wait()
    c2.wait()
    s1 = p1_v[0] + p1_v[1]
    s2 = p2_v[0]
    m1 = s1[:, :D] / jnp.maximum(s1[:, D:D + 1], 1.0)
    m2 = s2[:, :D] / jnp.maximum(s2[:, D:D + 1], 1.0)
    ft = jnp.concatenate([master_ref[...], m1, m2], axis=1)
    h = jax.nn.softplus(
        jnp.dot(ft, w1_ref[...], preferred_element_type=jnp.float32,
                precision=lax.Precision.HIGHEST) + b1_ref[...])
    h = jax.nn.softplus(
        jnp.dot(h, w2_ref[...], preferred_element_type=jnp.float32,
                precision=lax.Precision.HIGHEST) + b2_ref[...])
    out_ref[...] = (
        jnp.dot(h, w3_ref[...], preferred_element_type=jnp.float32,
                precision=lax.Precision.HIGHEST) + b3_ref[...])


def _mlp(master, p1, p2, W1, b1, W2, b2, W3, b3):
    n_blk = N_ATOM // BLK
    return pl.pallas_call(
        _mlp_body,
        grid=(n_blk,),
        in_specs=[
            pl.BlockSpec((BLK, D), lambda i: (i, 0)),
            pl.BlockSpec(memory_space=pltpu.MemorySpace.HBM),
            pl.BlockSpec(memory_space=pltpu.MemorySpace.HBM),
            pl.BlockSpec((3 * D, 64), lambda i: (0, 0)),
            pl.BlockSpec((1, 64), lambda i: (0, 0)),
            pl.BlockSpec((64, 64), lambda i: (0, 0)),
            pl.BlockSpec((1, 64), lambda i: (0, 0)),
            pl.BlockSpec((64, 32), lambda i: (0, 0)),
            pl.BlockSpec((1, 32), lambda i: (0, 0)),
        ],
        out_specs=pl.BlockSpec((BLK, 32), lambda i: (i, 0)),
        out_shape=jax.ShapeDtypeStruct((N_ATOM, 32), jnp.float32),
        scratch_shapes=[
            pltpu.VMEM((NC, BLK, DW), jnp.float32),
            pltpu.VMEM((1, BLK, DW), jnp.float32),
            pltpu.SemaphoreType.DMA,
            pltpu.SemaphoreType.DMA,
        ],
    )(master, p1, p2, W1, b1.reshape(1, -1), W2, b2.reshape(1, -1),
      W3, b3.reshape(1, -1))


def kernel(master_feats, bond_feats, global_feats, edge_index_bond,
           src_global, dst_global, W1, b1, W2, b2, W3, b3):
    ones16_b = jnp.ones((N_BOND, 16), jnp.float32)
    ones16_g = jnp.ones((N_GLOBAL, 16), jnp.float32)
    bond_ext = jnp.concatenate([bond_feats, ones16_b], axis=1)
    glob_ext = jnp.concatenate([global_feats, ones16_g], axis=1)
    zeros = jnp.zeros((ACC_ROWS, DW), jnp.float32)

    nch2 = _ceil_div(E_GLOBAL, NS * CHUNK)
    nch2 += nch2 % 2
    slab1 = _edge_chunks(edge_index_bond[0], edge_index_bond[1],
                         NS * (NCH_C0 + NCH_C1))
    slab2 = _edge_chunks(src_global, dst_global, NS * nch2)

    p1, p2 = _make_segsum(NCH_C0, NCH_C1, nch2)(bond_ext, glob_ext,
                                                slab1, slab2, zeros)

    return _mlp(master_feats, p1, p2, W1, b1, W2, b2, W3, b3)


# speedup vs baseline: 1.1988x; 1.1988x over previous
import functools

import jax
import jax.numpy as jnp
from jax import lax
from jax.experimental import pallas as pl
from jax.experimental.pallas import tpu as pltpu
from jax.experimental.pallas import tpu_sc as plsc

N_ATOM = 10000
N_BOND = 10000
N_GLOBAL = 64
E_BOND = 320000
E_GLOBAL = 10000
D = 128
DW = D + 16
ACC_ROWS = 10112
NC, NS = 2, 16
NW = NC * NS
CHUNK = 128
NCH_C0 = 102
NCH_C1 = 56
ROWS_PER_SUB = ACC_ROWS // NS


def _ceil_div(a, b):
    return (a + b - 1) // b


def _even_chunks(n_edges):
    n = _ceil_div(n_edges, NW * CHUNK)
    return n + (n % 2)


@functools.lru_cache(maxsize=None)
def _make_segsum(nch1a, nch1b, nch2):
    mesh = plsc.VectorSubcoreMesh(core_axis_name="c", subcore_axis_name="s")

    @functools.partial(
        pl.kernel,
        out_type=(jax.ShapeDtypeStruct((NC, ACC_ROWS, DW), jnp.float32),
                  jax.ShapeDtypeStruct((1, ACC_ROWS, DW), jnp.float32)),
        mesh=mesh,
        scratch_types=[
            pltpu.VMEM((2, CHUNK), jnp.int32),
            pltpu.VMEM((2, CHUNK), jnp.int32),
            pltpu.VMEM((CHUNK, DW), jnp.float32),
            pltpu.VMEM((CHUNK, DW), jnp.float32),
            pltpu.VMEM_SHARED((ACC_ROWS, DW), jnp.float32),
            pltpu.SemaphoreType.DMA,
            pltpu.SemaphoreType.DMA,
            pltpu.SemaphoreType.DMA,
        ],
        compiler_params=pltpu.CompilerParams(use_tc_tiling_on_sc=False),
    )
    def segsum(tab1_hbm, tab2_hbm, slab1_hbm, slab2_hbm, zeros_hbm,
               out1_hbm, out2_hbm,
               idx_a, idx_b, rows_a, rows_b, acc, sem_a, sem_b, sem_ib):
        cid = lax.axis_index("c")
        sid = lax.axis_index("s")
        r0 = sid * ROWS_PER_SUB

        def phase(table_hbm, slab_hbm, out_hbm, n_chunks, out_idx, base):
            pltpu.sync_copy(zeros_hbm.at[pl.ds(r0, ROWS_PER_SUB)],
                            acc.at[pl.ds(r0, ROWS_PER_SUB)])
            plsc.subcore_barrier()
            pltpu.sync_copy(slab_hbm.at[base], idx_a)
            pltpu.async_copy(table_hbm.at[idx_a.at[0]], rows_a, sem_a)
            pltpu.async_copy(slab_hbm.at[base + 1], idx_b, sem_ib)

            @pl.loop(0, n_chunks, step=2)
            def _(j):
                pltpu.make_async_copy(slab_hbm.at[base + j + 1], idx_b,
                                      sem_ib).wait()
                pltpu.async_copy(table_hbm.at[idx_b.at[0]], rows_b, sem_b)
                pltpu.make_async_copy(table_hbm.at[idx_a.at[0]], rows_a,
                                      sem_a).wait()
                pltpu.sync_copy(rows_a, acc.at[idx_a.at[1]], add=True)

                @pl.when(j + 2 < n_chunks)
                def _():
                    pltpu.sync_copy(slab_hbm.at[base + j + 2], idx_a)
                    pltpu.async_copy(table_hbm.at[idx_a.at[0]], rows_a, sem_a)

                pltpu.make_async_copy(table_hbm.at[idx_b.at[0]], rows_b,
                                      sem_b).wait()
                pltpu.sync_copy(rows_b, acc.at[idx_b.at[1]], add=True)

                @pl.when(j + 3 < n_chunks)
                def _():
                    pltpu.async_copy(slab_hbm.at[base + j + 3], idx_b, sem_ib)

            plsc.subcore_barrier()
            pltpu.sync_copy(acc.at[pl.ds(r0, ROWS_PER_SUB)],
                            out_hbm.at[out_idx].at[pl.ds(r0, ROWS_PER_SUB)])

        @pl.when(cid == 0)
        def _():
            phase(tab1_hbm, slab1_hbm, out1_hbm, nch1a, 0, sid * nch1a)
            phase(tab2_hbm, slab2_hbm, out2_hbm, nch2, 0, sid * nch2)

        @pl.when(cid == 1)
        def _():
            phase(tab1_hbm, slab1_hbm, out1_hbm, nch1b, 1,
                  NS * nch1a + sid * nch1b)

    return segsum


def _edge_chunks(src, dst, n_chunks_total):
    pad = n_chunks_total * CHUNK - src.shape[0]
    dummy = N_ATOM + jnp.arange(pad, dtype=jnp.int32) % (ACC_ROWS - N_ATOM)
    src_p = jnp.concatenate([src, jnp.zeros((pad,), jnp.int32)])
    dst_p = jnp.concatenate([dst, dummy])
    return jnp.stack([src_p.reshape(-1, CHUNK),
                      dst_p.reshape(-1, CHUNK)], axis=1)


BLK = 2000


def _mlp_body(master_ref, p1_ref, p2_ref, w1_ref, b1_ref, w2_ref, b2_ref,
              w3_ref, b3_ref, out_ref):
    s1 = p1_ref[0] + p1_ref[1]
    s2 = p2_ref[0]
    m1 = s1[:, :D] / jnp.maximum(s1[:, D:D + 1], 1.0)
    m2 = s2[:, :D] / jnp.maximum(s2[:, D:D + 1], 1.0)
    ft = jnp.concatenate([master_ref[...], m1, m2], axis=1)
    h = jax.nn.softplus(
        jnp.dot(ft, w1_ref[...], preferred_element_type=jnp.float32) + b1_ref[...])
    h = jax.nn.softplus(
        jnp.dot(h, w2_ref[...], preferred_element_type=jnp.float32) + b2_ref[...])
    out_ref[...] = (
        jnp.dot(h, w3_ref[...], preferred_element_type=jnp.float32) + b3_ref[...])


def _mlp(master, p1, p2, W1, b1, W2, b2, W3, b3):
    n_blk = N_ATOM // BLK
    return pl.pallas_call(
        _mlp_body,
        grid=(n_blk,),
        in_specs=[
            pl.BlockSpec((BLK, D), lambda i: (i, 0)),
            pl.BlockSpec((NC, BLK, DW), lambda i: (0, i, 0)),
            pl.BlockSpec((1, BLK, DW), lambda i: (0, i, 0)),
            pl.BlockSpec((3 * D, 64), lambda i: (0, 0)),
            pl.BlockSpec((1, 64), lambda i: (0, 0)),
            pl.BlockSpec((64, 64), lambda i: (0, 0)),
            pl.BlockSpec((1, 64), lambda i: (0, 0)),
            pl.BlockSpec((64, 32), lambda i: (0, 0)),
            pl.BlockSpec((1, 32), lambda i: (0, 0)),
        ],
        out_specs=pl.BlockSpec((BLK, 32), lambda i: (i, 0)),
        out_shape=jax.ShapeDtypeStruct((N_ATOM, 32), jnp.float32),
    )(master, p1, p2, W1, b1.reshape(1, -1), W2, b2.reshape(1, -1),
      W3, b3.reshape(1, -1))


def kernel(master_feats, bond_feats, global_feats, edge_index_bond,
           src_global, dst_global, W1, b1, W2, b2, W3, b3):
    ones16_b = jnp.ones((N_BOND, 16), jnp.float32)
    ones16_g = jnp.ones((N_GLOBAL, 16), jnp.float32)
    bond_ext = jnp.concatenate([bond_feats, ones16_b], axis=1)
    glob_ext = jnp.concatenate([global_feats, ones16_g], axis=1)
    zeros = jnp.zeros((ACC_ROWS, DW), jnp.float32)

    nch2 = _ceil_div(E_GLOBAL, NS * CHUNK)
    nch2 += nch2 % 2
    slab1 = _edge_chunks(edge_index_bond[0], edge_index_bond[1],
                         NS * (NCH_C0 + NCH_C1))
    slab2 = _edge_chunks(src_global, dst_global, NS * nch2)

    p1, p2 = _make_segsum(NCH_C0, NCH_C1, nch2)(bond_ext, glob_ext,
                                                slab1, slab2, zeros)

    return _mlp(master_feats, p1, p2, W1, b1, W2, b2, W3, b3)
